# Initial kernel scaffold; baseline (speedup 1.0000x reference)
import jax
import jax.numpy as jnp
from jax import lax
from jax.experimental import pallas as pl
from jax.experimental.pallas import tpu as pltpu
from jax.experimental.pallas import tpu_sc as plsc

B, F, H, D = 4096, 26, 20, 32
ROWS = B * F
NC, NS = 2, 16
NW = NC * NS
ROWS_W = ROWS // NW
CHUNK_ROWS = 32
CHUNK_IDX = CHUNK_ROWS * H
GATHER = 128
N_GATHER = CHUNK_IDX // GATHER
N_CHUNKS = ROWS_W // CHUNK_ROWS
IDX_ROWS_W = ROWS_W * H // GATHER


def _body(idx_hbm, table_hbm, out_hbm, idx_v, rows_v, out_v, sem):
    wid = lax.axis_index("s") * NC + lax.axis_index("c")
    base = wid * ROWS_W
    pltpu.sync_copy(idx_hbm.at[pl.ds(wid * IDX_ROWS_W, IDX_ROWS_W)], idx_v)

    def chunk_body(g, carry):
        handles = []
        for k in range(N_GATHER):
            handles.append(pltpu.async_copy(
                table_hbm.at[idx_v.at[g * N_GATHER + k]],
                rows_v.at[pl.ds(k * GATHER, GATHER)],
                sem))
        for h in handles:
            h.wait()
        for i in range(CHUNK_ROWS):
            rb = i * H
            lo = rows_v[rb, pl.ds(0, 16)]
            hi = rows_v[rb, pl.ds(16, 16)]
            for j in range(1, H):
                lo = lo + rows_v[rb + j, pl.ds(0, 16)]
                hi = hi + rows_v[rb + j, pl.ds(16, 16)]
            out_v[i, pl.ds(0, 16)] = lo
            out_v[i, pl.ds(16, 16)] = hi
        pltpu.sync_copy(out_v,
                        out_hbm.at[pl.ds(base + g * CHUNK_ROWS, CHUNK_ROWS)])
        return carry

    lax.fori_loop(0, N_CHUNKS, chunk_body, 0)


_kern = pl.kernel(
    _body,
    out_type=jax.ShapeDtypeStruct((ROWS, D), jnp.float32),
    mesh=plsc.VectorSubcoreMesh(core_axis_name="c", subcore_axis_name="s"),
    scratch_types=[
        pltpu.VMEM((IDX_ROWS_W, GATHER), jnp.int32),
        pltpu.VMEM((CHUNK_IDX, D), jnp.float32),
        pltpu.VMEM((CHUNK_ROWS, D), jnp.float32),
        pltpu.SemaphoreType.DMA,
    ],
)


@jax.jit
def kernel(x, emb_weight):
    idx = x.astype(jnp.int32).reshape(NW * IDX_ROWS_W, GATHER)
    out = _kern(idx, emb_weight)
    return out.reshape(B, F, D)


# speedup vs baseline: 14.7145x; 14.7145x over previous
import jax
import jax.numpy as jnp
from jax import lax
from jax.experimental import pallas as pl
from jax.experimental.pallas import tpu as pltpu
from jax.experimental.pallas import tpu_sc as plsc

B, F, H, D = 4096, 26, 20, 32
ROWS = B * F
NC, NS = 2, 16
NW = NC * NS
ROWS_W = ROWS // NW
CHUNK_ROWS = 32
CHUNK_IDX = CHUNK_ROWS * H
GATHER = 128
N_GATHER = CHUNK_IDX // GATHER
N_CHUNKS = ROWS_W // CHUNK_ROWS
IDX_ROWS_W = ROWS_W * H // GATHER


def _body(idx_hbm, table_hbm, out_hbm, idx_v, rows_v, out_v, sem):
    wid = lax.axis_index("s") * NC + lax.axis_index("c")
    base = wid * ROWS_W
    pltpu.sync_copy(idx_hbm.at[pl.ds(wid * IDX_ROWS_W, IDX_ROWS_W)], idx_v)

    def chunk_body(g, carry):
        handles = []
        for k in range(N_GATHER):
            handles.append(pltpu.async_copy(
                table_hbm.at[idx_v.at[g * N_GATHER + k]],
                rows_v.at[pl.ds(k * GATHER, GATHER)],
                sem))
        for h in handles:
            h.wait()
        for i in range(CHUNK_ROWS):
            rb = i * H
            lo = rows_v[rb, pl.ds(0, 16)]
            hi = rows_v[rb, pl.ds(16, 16)]
            for j in range(1, H):
                lo = lo + rows_v[rb + j, pl.ds(0, 16)]
                hi = hi + rows_v[rb + j, pl.ds(16, 16)]
            out_v[i, pl.ds(0, 16)] = lo
            out_v[i, pl.ds(16, 16)] = hi
        pltpu.sync_copy(out_v,
                        out_hbm.at[pl.ds(base + g * CHUNK_ROWS, CHUNK_ROWS)])
        return carry

    lax.fori_loop(0, N_CHUNKS, chunk_body, 0)


_kern = pl.kernel(
    _body,
    out_type=jax.ShapeDtypeStruct((ROWS, D), jnp.float32),
    mesh=plsc.VectorSubcoreMesh(core_axis_name="c", subcore_axis_name="s"),
    compiler_params=pltpu.CompilerParams(use_tc_tiling_on_sc=False),
    scratch_types=[
        pltpu.VMEM((IDX_ROWS_W, GATHER), jnp.int32),
        pltpu.VMEM((CHUNK_IDX, D), jnp.float32),
        pltpu.VMEM((CHUNK_ROWS, D), jnp.float32),
        pltpu.SemaphoreType.DMA,
    ],
)


@jax.jit
def kernel(x, emb_weight):
    idx = x.astype(jnp.int32).reshape(NW * IDX_ROWS_W, GATHER)
    out = _kern(idx, emb_weight)
    return out.reshape(B, F, D)


# speedup vs baseline: 29.1645x; 1.9820x over previous
import jax
import jax.numpy as jnp
from jax import lax
from jax.experimental import pallas as pl
from jax.experimental.pallas import tpu as pltpu
from jax.experimental.pallas import tpu_sc as plsc

B, F, H, D = 4096, 26, 20, 32
NC, NS = 2, 16
NW = NC * NS
B_W = B // NW
CHUNK_IDX = F * H
GATHER = 104
N_GATHER = CHUNK_IDX // GATHER
IDX_ROWS_W = B_W * CHUNK_IDX // GATHER


def _body(idx_hbm, table_hbm, out_hbm, idx_v, rows0, rows1, out0, out1,
          gsem0, gsem1, osem0, osem1):
    wid = lax.axis_index("s") * NC + lax.axis_index("c")
    bbase = wid * B_W
    rows = (rows0, rows1)
    outs = (out0, out1)
    gsems = (gsem0, gsem1)
    osems = (osem0, osem1)

    pltpu.sync_copy(idx_hbm.at[pl.ds(wid * IDX_ROWS_W, IDX_ROWS_W)], idx_v)

    def issue(g, b):
        for k in range(N_GATHER):
            pltpu.async_copy(
                table_hbm.at[idx_v.at[g * N_GATHER + k]],
                rows[b].at[pl.ds(k * GATHER, GATHER)],
                gsems[b])

    def wait_gather(b):
        pltpu.make_async_copy(
            table_hbm.at[pl.ds(0, CHUNK_IDX)], rows[b], gsems[b]).wait()

    def wait_out(b):
        pltpu.make_async_copy(
            outs[b], out_hbm.at[pl.ds(bbase, 1)], osems[b]).wait()

    def compute(b):
        @plsc.parallel_loop(0, F, 1, unroll=4)
        def _row(i):
            rb = i * H
            for half in range(2):
                accs = [rows[b][rb + j, pl.ds(half * 16, 16)]
                        for j in range(4)]
                for j in range(4, H):
                    accs[j % 4] = accs[j % 4] + rows[b][rb + j,
                                                        pl.ds(half * 16, 16)]
                outs[b][0, i, pl.ds(half * 16, 16)] = (
                    (accs[0] + accs[1]) + (accs[2] + accs[3]))

    issue(0, 0)

    def step(p, carry):
        for b in range(2):
            g = 2 * p + b
            nb = 1 - b

            @pl.when(g + 1 < B_W)
            def _():
                issue(g + 1, nb)

            wait_gather(b)

            @pl.when(g >= 2)
            def _():
                wait_out(b)

            compute(b)
            pltpu.async_copy(
                outs[b], out_hbm.at[pl.ds(bbase + g, 1)], osems[b])
        return carry

    lax.fori_loop(0, B_W // 2, step, 0)
    wait_out(0)
    wait_out(1)


_kern = pl.kernel(
    _body,
    out_type=jax.ShapeDtypeStruct((B, F, D), jnp.float32),
    mesh=plsc.VectorSubcoreMesh(core_axis_name="c", subcore_axis_name="s"),
    compiler_params=pltpu.CompilerParams(use_tc_tiling_on_sc=False),
    scratch_types=[
        pltpu.VMEM((IDX_ROWS_W, GATHER), jnp.int32),
        pltpu.VMEM((CHUNK_IDX, D), jnp.float32),
        pltpu.VMEM((CHUNK_IDX, D), jnp.float32),
        pltpu.VMEM((1, F, D), jnp.float32),
        pltpu.VMEM((1, F, D), jnp.float32),
        pltpu.SemaphoreType.DMA,
        pltpu.SemaphoreType.DMA,
        pltpu.SemaphoreType.DMA,
        pltpu.SemaphoreType.DMA,
    ],
)


@jax.jit
def kernel(x, emb_weight):
    idx = x.astype(jnp.int32).reshape(NW * IDX_ROWS_W, GATHER)
    return _kern(idx, emb_weight)


# speedup vs baseline: 29.4468x; 1.0097x over previous
import jax
import jax.numpy as jnp
from jax import lax
from jax.experimental import pallas as pl
from jax.experimental.pallas import tpu as pltpu
from jax.experimental.pallas import tpu_sc as plsc

B, F, H, D = 4096, 26, 20, 32
NC, NS = 2, 16
NW = NC * NS
B_W = B // NW
CHUNK_IDX = F * H
GATHER = 104
N_GATHER = CHUNK_IDX // GATHER
IDX_ROWS_W = B_W * CHUNK_IDX // GATHER


def _body(idx_hbm, table_hbm, out_hbm, idx_v, rows0, rows1, out0, out1,
          gsem0, gsem1, osem0, osem1):
    wid = lax.axis_index("s") * NC + lax.axis_index("c")
    bbase = wid * B_W
    rows = (rows0, rows1)
    outs = (out0, out1)
    gsems = (gsem0, gsem1)
    osems = (osem0, osem1)

    pltpu.sync_copy(idx_hbm.at[pl.ds(wid * IDX_ROWS_W, IDX_ROWS_W)], idx_v)

    def issue(g, b):
        for k in range(N_GATHER):
            pltpu.async_copy(
                table_hbm.at[idx_v.at[g * N_GATHER + k]],
                rows[b].at[pl.ds(k * GATHER, GATHER)],
                gsems[b])

    def wait_gather(b):
        pltpu.make_async_copy(
            table_hbm.at[pl.ds(0, CHUNK_IDX)], rows[b], gsems[b]).wait()

    def wait_out(b):
        pltpu.make_async_copy(
            outs[b], out_hbm.at[pl.ds(bbase, 1)], osems[b]).wait()

    def compute(b):
        @plsc.parallel_loop(0, F, 1, unroll=13)
        def _row(i):
            rb = i * H
            for half in range(2):
                accs = [rows[b][rb + j, pl.ds(half * 16, 16)]
                        for j in range(4)]
                for j in range(4, H):
                    accs[j % 4] = accs[j % 4] + rows[b][rb + j,
                                                        pl.ds(half * 16, 16)]
                outs[b][0, i, pl.ds(half * 16, 16)] = (
                    (accs[0] + accs[1]) + (accs[2] + accs[3]))

    issue(0, 0)

    def step(p, carry):
        for b in range(2):
            g = 2 * p + b
            nb = 1 - b

            @pl.when(g + 1 < B_W)
            def _():
                issue(g + 1, nb)

            wait_gather(b)

            @pl.when(g >= 2)
            def _():
                wait_out(b)

            compute(b)
            pltpu.async_copy(
                outs[b], out_hbm.at[pl.ds(bbase + g, 1)], osems[b])
        return carry

    lax.fori_loop(0, B_W // 2, step, 0)
    wait_out(0)
    wait_out(1)


_kern = pl.kernel(
    _body,
    out_type=jax.ShapeDtypeStruct((B, F, D), jnp.float32),
    mesh=plsc.VectorSubcoreMesh(core_axis_name="c", subcore_axis_name="s"),
    compiler_params=pltpu.CompilerParams(use_tc_tiling_on_sc=False),
    scratch_types=[
        pltpu.VMEM((IDX_ROWS_W, GATHER), jnp.int32),
        pltpu.VMEM((CHUNK_IDX, D), jnp.float32),
        pltpu.VMEM((CHUNK_IDX, D), jnp.float32),
        pltpu.VMEM((1, F, D), jnp.float32),
        pltpu.VMEM((1, F, D), jnp.float32),
        pltpu.SemaphoreType.DMA,
        pltpu.SemaphoreType.DMA,
        pltpu.SemaphoreType.DMA,
        pltpu.SemaphoreType.DMA,
    ],
)


@jax.jit
def kernel(x, emb_weight):
    idx = x.astype(jnp.int32).reshape(NW * IDX_ROWS_W, GATHER)
    return _kern(idx, emb_weight)
